# baseline (speedup 1.0000x reference)
import jax
import jax.numpy as jnp
import numpy as np
from jax.experimental import pallas as pl
from jax.experimental.pallas import tpu as pltpu

B, S, C = 4, 19, 192
H = W = 224
HG = WG = 56
OUT = 512

GSUB = 4 * W
WGP = 64
NGRP = (H * W) // GSUB
CB = 48
NSTEP = C // CB


def _pool_matrix() -> np.ndarray:
    j = np.arange(GSUB)
    pw = np.zeros((GSUB, WGP), np.float32)
    pw[j, (j % W) // 4] = 1.0
    return pw


def _tok_kernel(codes_ref, mseg_ref, pw_ref, fcw_ref, fcb_ref, out_ref,
                sums_ref, area_ref):
    cb = pl.program_id(1)

    @pl.when(cb == 0)
    def _init():
        sums_ref[...] = jnp.zeros_like(sums_ref)
        area_ref[...] = jnp.zeros_like(area_ref)

    x = codes_ref[0]
    c0 = cb * CB
    for j in range(NGRP):
        m = (mseg_ref[0, j] != 0).astype(jnp.float32)
        xj = x[:, j * GSUB:(j + 1) * GSUB]
        yp = jnp.dot(xj, pw_ref[...], preferred_element_type=jnp.float32)
        sums_ref[pl.ds(c0, CB)] += jnp.dot(
            yp, m, preferred_element_type=jnp.float32)

        @pl.when(cb == 0)
        def _area():
            area_ref[...] += jnp.sum(m, axis=0, keepdims=True)

    @pl.when(cb == NSTEP - 1)
    def _fin():
        area = area_ref[...]
        inv = jnp.where(area > 0, 1.0 / (16.0 * jnp.maximum(area, 1.0)), 0.0)
        vec = sums_ref[...] * inv
        out_ref[0] = (jnp.dot(fcw_ref[...], vec,
                              preferred_element_type=jnp.float32)
                      + fcb_ref[...])


@jax.jit
def kernel(codes, segmap, fc_w, fc_b):
    codes3 = codes.reshape(B, C, H * W)
    mseg = (segmap.reshape(B, S, HG, WG)
            .transpose(0, 2, 3, 1))
    mseg = jnp.pad(mseg, ((0, 0), (0, 0), (0, WGP - WG), (0, 0)))
    pw = jnp.asarray(_pool_matrix())
    fcb2 = fc_b.reshape(OUT, 1)

    out_t = pl.pallas_call(
        _tok_kernel,
        grid=(B, NSTEP),
        in_specs=[
            pl.BlockSpec((1, CB, H * W), lambda b, c: (b, c, 0)),
            pl.BlockSpec((1, NGRP, WGP, S), lambda b, c: (b, 0, 0, 0)),
            pl.BlockSpec((GSUB, WGP), lambda b, c: (0, 0)),
            pl.BlockSpec((OUT, C), lambda b, c: (0, 0)),
            pl.BlockSpec((OUT, 1), lambda b, c: (0, 0)),
        ],
        out_specs=pl.BlockSpec((1, OUT, S), lambda b, c: (b, 0, 0)),
        out_shape=jax.ShapeDtypeStruct((B, OUT, S), jnp.float32),
        scratch_shapes=[
            pltpu.VMEM((C, S), jnp.float32),
            pltpu.VMEM((1, S), jnp.float32),
        ],
    )(codes3, mseg, pw, fc_w, fcb2)
    return out_t.transpose(0, 2, 1)


# speedup vs baseline: 1.8449x; 1.8449x over previous
import jax
import jax.numpy as jnp
import numpy as np
from jax.experimental import pallas as pl
from jax.experimental.pallas import tpu as pltpu

B, S, C = 4, 19, 192
H = W = 224
HG = WG = 56
OUT = 512

GSUB = 8 * W
WGP = 64
MC = 2 * WGP
NSLAB = 7
KBLK = NSLAB * GSUB
NSTEP = (H * W) // KBLK


def _pool_matrix() -> np.ndarray:
    j = np.arange(GSUB)
    pw = np.zeros((GSUB, MC), np.float32)
    pw[j, (j // W // 4) * WGP + (j % W) // 4] = 1.0
    return pw


def _tok_kernel(codes_ref, mseg_ref, pw_ref, fcw_ref, fcb_ref, out_ref,
                sums_ref, area_ref):
    hb = pl.program_id(1)

    @pl.when(hb == 0)
    def _init():
        sums_ref[...] = jnp.zeros_like(sums_ref)
        area_ref[...] = jnp.zeros_like(area_ref)

    x = codes_ref[0]
    for j in range(NSLAB):
        m = mseg_ref[0, 0, j]
        xj = x[:, j * GSUB:(j + 1) * GSUB]
        yp = jnp.dot(xj, pw_ref[...], preferred_element_type=jnp.float32)
        sums_ref[...] += jnp.dot(yp, m, preferred_element_type=jnp.float32)
        area_ref[...] += jnp.sum(m, axis=0, keepdims=True)

    @pl.when(hb == NSTEP - 1)
    def _fin():
        area = area_ref[...]
        inv = jnp.where(area > 0, 1.0 / (16.0 * jnp.maximum(area, 1.0)), 0.0)
        vec = sums_ref[...] * inv
        out_ref[0] = (jnp.dot(fcw_ref[...], vec,
                              preferred_element_type=jnp.float32)
                      + fcb_ref[...])


@jax.jit
def kernel(codes, segmap, fc_w, fc_b):
    codes3 = codes.reshape(B, C, H * W)
    mseg = (segmap.reshape(B, S, HG, WG)
            .transpose(0, 2, 3, 1))
    mseg = jnp.pad(mseg, ((0, 0), (0, 0), (0, WGP - WG), (0, 0)))
    mseg = mseg.reshape(B, NSTEP, NSLAB, MC, S)
    pw = jnp.asarray(_pool_matrix())
    fcb2 = fc_b.reshape(OUT, 1)

    out_t = pl.pallas_call(
        _tok_kernel,
        grid=(B, NSTEP),
        in_specs=[
            pl.BlockSpec((1, C, KBLK), lambda b, h: (b, 0, h)),
            pl.BlockSpec((1, 1, NSLAB, MC, S), lambda b, h: (b, h, 0, 0, 0)),
            pl.BlockSpec((GSUB, MC), lambda b, h: (0, 0)),
            pl.BlockSpec((OUT, C), lambda b, h: (0, 0)),
            pl.BlockSpec((OUT, 1), lambda b, h: (0, 0)),
        ],
        out_specs=pl.BlockSpec((1, OUT, S), lambda b, h: (b, 0, 0)),
        out_shape=jax.ShapeDtypeStruct((B, OUT, S), jnp.float32),
        scratch_shapes=[
            pltpu.VMEM((C, S), jnp.float32),
            pltpu.VMEM((1, S), jnp.float32),
        ],
    )(codes3, mseg, pw, fc_w, fcb2)
    return out_t.transpose(0, 2, 1)
